# baseline (speedup 1.0000x reference)
import functools

import jax
import jax.numpy as jnp
from jax import lax
from jax.experimental import pallas as pl
from jax.experimental.pallas import tpu as pltpu
from jax.experimental.pallas import tpu_sc as plsc

B = 16384
D = 32
SLAB = 128
NC = 2
NS = 16
NW = NC * NS
BPW = B // NW
CH = 128
NCH = BPW // CH


def _sc_gather(us_hbm, is_hbm, gu_hbm, gi_hbm, mu_hbm, mi_hbm,
               gu_out, gi_out, mu_out, mi_out,
               uidx_v, iidx_v, gu_v, gi_v, mu_v, mi_v, gsem, osem):
    wid = lax.axis_index("s") * NC + lax.axis_index("c")
    pltpu.sync_copy(us_hbm.at[pl.ds(wid * NCH, NCH)], uidx_v)
    pltpu.sync_copy(is_hbm.at[pl.ds(wid * NCH, NCH)], iidx_v)
    outs = []
    for c in range(NCH):
        gathers = [
            pltpu.async_copy(gu_hbm.at[uidx_v.at[c]], gu_v, gsem),
            pltpu.async_copy(gi_hbm.at[iidx_v.at[c]], gi_v, gsem),
            pltpu.async_copy(mu_hbm.at[uidx_v.at[c]], mu_v, gsem),
            pltpu.async_copy(mi_hbm.at[iidx_v.at[c]], mi_v, gsem),
        ]
        for g in gathers:
            g.wait()
        rows = pl.ds(wid * BPW + c * CH, CH)
        outs = [
            pltpu.async_copy(gu_v, gu_out.at[rows], osem),
            pltpu.async_copy(gi_v, gi_out.at[rows], osem),
            pltpu.async_copy(mu_v, mu_out.at[rows], osem),
            pltpu.async_copy(mi_v, mi_out.at[rows], osem),
        ]
        if c < NCH - 1:
            for o in outs:
                o.wait()
    for o in outs:
        o.wait()


@functools.cache
def _sc_gather_call():
    return functools.partial(
        pl.kernel,
        mesh=plsc.VectorSubcoreMesh(core_axis_name="c", subcore_axis_name="s"),
        out_type=[jax.ShapeDtypeStruct((B, SLAB), jnp.float32)] * 4,
        scratch_types=[
            pltpu.VMEM((NCH, CH), jnp.int32),
            pltpu.VMEM((NCH, CH), jnp.int32),
            pltpu.VMEM((CH, SLAB), jnp.float32),
            pltpu.VMEM((CH, SLAB), jnp.float32),
            pltpu.VMEM((CH, SLAB), jnp.float32),
            pltpu.VMEM((CH, SLAB), jnp.float32),
            pltpu.SemaphoreType.DMA,
            pltpu.SemaphoreType.DMA,
        ],
        compiler_params=pltpu.CompilerParams(use_tc_tiling_on_sc=True),
    )(_sc_gather)


BT = 2048


def _tc_body(gu, gi, mu, mi, msku, mski,
             w1u, w1i, b1, w2, b2, w3, b3, wog, woh, bo, out):
    def ext(slab, mask):
        p = slab[...] * mask[...]
        return (p[:, 0:32] + p[:, 32:64]) + (p[:, 64:96] + p[:, 96:128])

    eu = ext(mu, msku)
    ei = ext(mi, mski)
    h = (jnp.dot(eu, w1u[...], preferred_element_type=jnp.float32)
         + jnp.dot(ei, w1i[...], preferred_element_type=jnp.float32)
         + b1[...])
    h = jnp.maximum(h, 0.0)
    h = jnp.maximum(jnp.dot(h, w2[...], preferred_element_type=jnp.float32) + b2[...], 0.0)
    h = jnp.maximum(jnp.dot(h, w3[...], preferred_element_type=jnp.float32) + b3[...], 0.0)
    g = ext(gu, msku) * ext(gi, mski)
    out[...] = (jnp.sum(g * wog[...], axis=1) + jnp.sum(h * woh[...], axis=1)
                + bo[0, 0])


def _tc_mlp(gu, gi, mu, mi, msku, mski,
            w1u, w1i, b1, w2, b2, w3, b3, wog, woh, bo):
    rows = lambda: pl.BlockSpec((BT, SLAB), lambda i: (i, 0))
    full = lambda a: pl.BlockSpec(a.shape, lambda i: (0,) * a.ndim)
    return pl.pallas_call(
        _tc_body,
        grid=(B // BT,),
        in_specs=[rows(), rows(), rows(), rows(), rows(), rows(),
                  full(w1u), full(w1i), full(b1), full(w2), full(b2),
                  full(w3), full(b3), full(wog), full(woh), full(bo)],
        out_specs=pl.BlockSpec((BT,), lambda i: (i,)),
        out_shape=jax.ShapeDtypeStruct((B,), jnp.float32),
    )(gu, gi, mu, mi, msku, mski,
      w1u, w1i, b1, w2, b2, w3, b3, wog, woh, bo)


def kernel(user_ids, item_ids, gmf_user_w, gmf_item_w, mlp_user_w, mlp_item_w,
           W1, b1, W2, b2, W3, b3, Wo, bo):
    uid = user_ids.astype(jnp.int32)
    iid = item_ids.astype(jnp.int32)
    uslab = (uid >> 2).reshape(B // CH, CH)
    islab = (iid >> 2).reshape(B // CH, CH)
    lanes = jnp.arange(SLAB, dtype=jnp.int32) >> 5
    msku = (lanes[None, :] == (uid & 3)[:, None]).astype(jnp.float32)
    mski = (lanes[None, :] == (iid & 3)[:, None]).astype(jnp.float32)
    tables = [t.reshape(-1, SLAB)
              for t in (gmf_user_w, gmf_item_w, mlp_user_w, mlp_item_w)]
    gu, gi, mu, mi = _sc_gather_call()(uslab, islab, *tables)
    w1u = W1[:, :D].T
    w1i = W1[:, D:].T
    out = _tc_mlp(gu, gi, mu, mi, msku, mski,
                  w1u, w1i, b1.reshape(1, -1),
                  W2.T, b2.reshape(1, -1),
                  W3.T, b3.reshape(1, -1),
                  Wo[:, :D], Wo[:, D:], bo.reshape(1, 1))
    return out


# speedup vs baseline: 1.1961x; 1.1961x over previous
import functools

import jax
import jax.numpy as jnp
from jax import lax
from jax.experimental import pallas as pl
from jax.experimental.pallas import tpu as pltpu
from jax.experimental.pallas import tpu_sc as plsc

B = 16384
D = 32
PW = 128
NC = 2
NS = 16
NW = NC * NS
BPW = B // NW
CH = 128
NCH = BPW // CH


def _sc_gather(us_hbm, is_hbm, z_hbm, zu_out, zi_out,
               uidx_v, iidx_v, zu_v, zi_v, gsem, osem):
    wid = lax.axis_index("s") * NC + lax.axis_index("c")
    pltpu.sync_copy(us_hbm.at[pl.ds(wid * NCH, NCH)], uidx_v)
    pltpu.sync_copy(is_hbm.at[pl.ds(wid * NCH, NCH)], iidx_v)
    outs = []
    for c in range(NCH):
        gathers = [
            pltpu.async_copy(z_hbm.at[uidx_v.at[c]], zu_v, gsem),
            pltpu.async_copy(z_hbm.at[iidx_v.at[c]], zi_v, gsem),
        ]
        for g in gathers:
            g.wait()
        rows = pl.ds(wid * BPW + c * CH, CH)
        outs = [
            pltpu.async_copy(zu_v, zu_out.at[rows], osem),
            pltpu.async_copy(zi_v, zi_out.at[rows], osem),
        ]
        if c < NCH - 1:
            for o in outs:
                o.wait()
    for o in outs:
        o.wait()


@functools.cache
def _sc_gather_call():
    return functools.partial(
        pl.kernel,
        mesh=plsc.VectorSubcoreMesh(core_axis_name="c", subcore_axis_name="s"),
        out_type=[jax.ShapeDtypeStruct((B, PW), jnp.float32)] * 2,
        scratch_types=[
            pltpu.VMEM((NCH, CH), jnp.int32),
            pltpu.VMEM((NCH, CH), jnp.int32),
            pltpu.VMEM((CH, PW), jnp.float32),
            pltpu.VMEM((CH, PW), jnp.float32),
            pltpu.SemaphoreType.DMA,
            pltpu.SemaphoreType.DMA,
        ],
        compiler_params=pltpu.CompilerParams(use_tc_tiling_on_sc=True),
    )(_sc_gather)


BT = 2048


def _tc_body(zu, zi, w1u, w1i, b1, w2, b2, w3, b3, wog, woh, bo, out):
    gu = zu[:, 0:32]
    mu = zu[:, 32:64]
    gi = zi[:, 64:96]
    mi = zi[:, 96:128]
    h = (jnp.dot(mu, w1u[...], preferred_element_type=jnp.float32)
         + jnp.dot(mi, w1i[...], preferred_element_type=jnp.float32)
         + b1[...])
    h = jnp.maximum(h, 0.0)
    h = jnp.maximum(jnp.dot(h, w2[...], preferred_element_type=jnp.float32) + b2[...], 0.0)
    h = jnp.maximum(jnp.dot(h, w3[...], preferred_element_type=jnp.float32) + b3[...], 0.0)
    g = gu * gi
    out[...] = (jnp.sum(g * wog[...], axis=1) + jnp.sum(h * woh[...], axis=1)
                + bo[0, 0])


def _tc_mlp(zu, zi, w1u, w1i, b1, w2, b2, w3, b3, wog, woh, bo):
    rows = lambda: pl.BlockSpec((BT, PW), lambda i: (i, 0))
    full = lambda a: pl.BlockSpec(a.shape, lambda i: (0,) * a.ndim)
    return pl.pallas_call(
        _tc_body,
        grid=(B // BT,),
        in_specs=[rows(), rows(),
                  full(w1u), full(w1i), full(b1), full(w2), full(b2),
                  full(w3), full(b3), full(wog), full(woh), full(bo)],
        out_specs=pl.BlockSpec((BT,), lambda i: (i,)),
        out_shape=jax.ShapeDtypeStruct((B,), jnp.float32),
    )(zu, zi, w1u, w1i, b1, w2, b2, w3, b3, wog, woh, bo)


def kernel(user_ids, item_ids, gmf_user_w, gmf_item_w, mlp_user_w, mlp_item_w,
           W1, b1, W2, b2, W3, b3, Wo, bo):
    uid2 = user_ids.astype(jnp.int32).reshape(B // CH, CH)
    iid2 = item_ids.astype(jnp.int32).reshape(B // CH, CH)
    z = jnp.concatenate([gmf_user_w, mlp_user_w, gmf_item_w, mlp_item_w],
                        axis=1)
    zu, zi = _sc_gather_call()(uid2, iid2, z)
    w1u = W1[:, :D].T
    w1i = W1[:, D:].T
    out = _tc_mlp(zu, zi,
                  w1u, w1i, b1.reshape(1, -1),
                  W2.T, b2.reshape(1, -1),
                  W3.T, b3.reshape(1, -1),
                  Wo[:, :D], Wo[:, D:], bo.reshape(1, 1))
    return out
